# baseline (speedup 1.0000x reference)
import functools

import jax
import jax.numpy as jnp
from jax import lax
from jax.experimental import pallas as pl
from jax.experimental.pallas import tpu as pltpu
from jax.experimental.pallas import tpu_sc as plsc


def _make_sc_broadcast(batch, seq_len, d_model, dtype):
    info = plsc.get_sparse_core_info()
    num_workers = info.num_cores * info.num_subcores
    rows_per_worker = seq_len // num_workers

    mesh = plsc.VectorSubcoreMesh(core_axis_name="c", subcore_axis_name="s")

    @functools.partial(
        pl.kernel,
        mesh=mesh,
        out_type=jax.ShapeDtypeStruct((batch, seq_len, d_model), dtype),
        scratch_types=[
            pltpu.SemaphoreType.DMA,
        ],
    )
    def sc_broadcast(table_hbm, out_hbm, sem):
        wid = lax.axis_index("s") * info.num_cores + lax.axis_index("c")
        base = wid * rows_per_worker
        copies = [
            pltpu.async_copy(
                table_hbm.at[pl.ds(base, rows_per_worker)],
                out_hbm.at[b, pl.ds(base, rows_per_worker)],
                sem,
            )
            for b in range(batch)
        ]
        for h in copies:
            h.wait()

    return sc_broadcast


def kernel(x, table):
    batch, seq_len, d_model = x.shape
    fn = _make_sc_broadcast(batch, seq_len, d_model, table.dtype)
    return fn(table)


# speedup vs baseline: 54.3922x; 54.3922x over previous
import functools

import jax
import jax.numpy as jnp
from jax import lax
from jax.experimental import pallas as pl
from jax.experimental.pallas import tpu as pltpu
from jax.experimental.pallas import tpu_sc as plsc

_NBUF = 3


def _make_sc_broadcast(batch, seq_len, d_model, dtype):
    info = plsc.get_sparse_core_info()
    num_workers = info.num_cores * info.num_subcores
    rows_per_worker = seq_len // num_workers
    chunk = min(32, rows_per_worker)
    num_chunks = rows_per_worker // chunk

    mesh = plsc.VectorSubcoreMesh(core_axis_name="c", subcore_axis_name="s")

    @functools.partial(
        pl.kernel,
        mesh=mesh,
        out_type=jax.ShapeDtypeStruct((batch, seq_len, d_model), dtype),
        scratch_types=(
            [pltpu.VMEM((chunk, d_model), dtype) for _ in range(_NBUF)]
            + [pltpu.SemaphoreType.DMA for _ in range(2 * _NBUF)]
        ),
    )
    def sc_broadcast(table_hbm, out_hbm, *refs):
        bufs = refs[:_NBUF]
        rsems = refs[_NBUF : 2 * _NBUF]
        wsems = refs[2 * _NBUF :]
        wid = lax.axis_index("s") * info.num_cores + lax.axis_index("c")
        base = wid * rows_per_worker

        def start_read(i):
            return pltpu.async_copy(
                table_hbm.at[pl.ds(base + i * chunk, chunk)],
                bufs[i % _NBUF],
                rsems[i % _NBUF],
            )

        reads = [None] * num_chunks
        writes = [None] * num_chunks
        reads[0] = start_read(0)
        for i in range(num_chunks):
            if i >= _NBUF - 1 and writes[i - (_NBUF - 1)] is not None:
                for h in writes[i - (_NBUF - 1)]:
                    h.wait()
                writes[i - (_NBUF - 1)] = None
            if i + 1 < num_chunks:
                reads[i + 1] = start_read(i + 1)
            reads[i].wait()
            writes[i] = [
                pltpu.async_copy(
                    bufs[i % _NBUF],
                    out_hbm.at[b, pl.ds(base + i * chunk, chunk)],
                    wsems[i % _NBUF],
                )
                for b in range(batch)
            ]
        for ws in writes:
            if ws is not None:
                for h in ws:
                    h.wait()

    return sc_broadcast


def kernel(x, table):
    batch, seq_len, d_model = x.shape
    fn = _make_sc_broadcast(batch, seq_len, d_model, table.dtype)
    return fn(table)
